# baseline (speedup 1.0000x reference)
import jax
import jax.numpy as jnp
from jax.experimental import pallas as pl
from jax.experimental.pallas import tpu as pltpu

_MARGIN = 0.2
_EPS = 1e-6

_BM = 256
_BN = 512


def _body(a_ref, bt_ref, tcol_ref, trow_ref, num_ref, cnt_ref,
          btbf_ref, cp_ref, cn_ref, nacc_ref, cacc_ref):
    i = pl.program_id(0)
    nsteps = pl.num_programs(0)
    d_dim = a_ref.shape[1]
    b_dim = bt_ref.shape[1]

    @pl.when(i == 0)
    def _init():
        bt = bt_ref[...]
        btbf_ref[...] = bt.astype(jnp.bfloat16)
        colterm = jnp.sum(bt * (bt - (2.0 * _EPS)), axis=0, keepdims=True)
        colterm = colterm + (d_dim * _EPS * _EPS)
        posm = tcol_ref[...] == 1
        cp_ref[...] = jnp.where(posm, colterm, -jnp.inf)
        cn_ref[...] = jnp.where(posm, jnp.inf, colterm)
        nacc_ref[...] = jnp.zeros_like(nacc_ref)
        cacc_ref[...] = jnp.zeros_like(cacc_ref)

    a = a_ref[...]
    am2 = (a * -2.0).astype(jnp.bfloat16)
    rowterm = jnp.sum(a * (a + (2.0 * _EPS)), axis=1, keepdims=True)

    acc_p = jnp.full((_BM, 128), -jnp.inf, jnp.float32)
    acc_n = jnp.full((_BM, 128), jnp.inf, jnp.float32)
    for c in range(b_dim // _BN):
        btc = btbf_ref[:, c * _BN:(c + 1) * _BN]
        t2 = jax.lax.dot_general(am2, btc, (((1,), (0,)), ((), ())),
                                 preferred_element_type=jnp.float32)
        cp = cp_ref[0:1, c * _BN:(c + 1) * _BN]
        cn = cn_ref[0:1, c * _BN:(c + 1) * _BN]
        tp = t2 + cp
        tn = t2 + cn
        for s in range(_BN // 128):
            acc_p = jnp.maximum(acc_p, tp[:, s * 128:(s + 1) * 128])
            acc_n = jnp.minimum(acc_n, tn[:, s * 128:(s + 1) * 128])

    msp = jnp.max(acc_p, axis=1, keepdims=True) + rowterm
    msn = jnp.min(acc_n, axis=1, keepdims=True) + rowterm
    dp = jnp.sqrt(jnp.maximum(msp, 0.0))
    dn = jnp.sqrt(jnp.maximum(msn, 0.0))
    w = (trow_ref[...] == 1).astype(jnp.float32)
    nacc_ref[...] += jnp.maximum(dp - dn + _MARGIN, 0.0) * w
    cacc_ref[...] += w

    @pl.when(i == nsteps - 1)
    def _fin():
        num_ref[...] = jnp.sum(nacc_ref[...], keepdims=True)
        cnt_ref[...] = jnp.sum(cacc_ref[...], keepdims=True)


def _run_block(emb1, bt, tcol, trow):
    b_rows, d_dim = emb1.shape
    b_dim = bt.shape[1]
    nb = b_rows // _BM

    num, cnt = pl.pallas_call(
        _body,
        grid=(nb,),
        in_specs=[
            pl.BlockSpec((_BM, d_dim), lambda i: (i, 0)),
            pl.BlockSpec((d_dim, b_dim), lambda i: (0, 0)),
            pl.BlockSpec((1, b_dim), lambda i: (0, 0)),
            pl.BlockSpec((_BM, 1), lambda i: (i, 0)),
        ],
        out_specs=[
            pl.BlockSpec((1, 1), lambda i: (0, 0)),
            pl.BlockSpec((1, 1), lambda i: (0, 0)),
        ],
        out_shape=[
            jax.ShapeDtypeStruct((1, 1), jnp.float32),
            jax.ShapeDtypeStruct((1, 1), jnp.float32),
        ],
        scratch_shapes=[
            pltpu.VMEM((d_dim, b_dim), jnp.bfloat16),
            pltpu.VMEM((1, b_dim), jnp.float32),
            pltpu.VMEM((1, b_dim), jnp.float32),
            pltpu.VMEM((_BM, 1), jnp.float32),
            pltpu.VMEM((_BM, 1), jnp.float32),
        ],
        compiler_params=pltpu.CompilerParams(
            dimension_semantics=("arbitrary",),
            vmem_limit_bytes=48 * 1024 * 1024,
        ),
    )(emb1, bt, tcol, trow)
    return num, cnt


def kernel(emb1, emb2, target):
    b_dim, d_dim = emb1.shape
    tgt = target.astype(jnp.int32)
    bt = emb2.T
    tcol = tgt.reshape(1, b_dim)
    trow = tgt.reshape(b_dim, 1)

    devs = jax.devices()
    n_shards = 2 if (len(devs) >= 2 and b_dim % (2 * _BM) == 0) else 1
    if n_shards > 1:
        mesh = jax.sharding.Mesh(devs[:n_shards], ("x",))
        P = jax.sharding.PartitionSpec
        num, cnt = jax.shard_map(
            _run_block,
            mesh=mesh,
            in_specs=(P("x", None), P(None, None), P(None, None), P("x", None)),
            out_specs=(P("x", None), P("x", None)),
            check_vma=False,
        )(emb1, bt, tcol, trow)
        return jnp.sum(num) / jnp.sum(cnt)
    num, cnt = _run_block(emb1, bt, tcol, trow)
    return num[0, 0] / cnt[0, 0]


# speedup vs baseline: 7.7682x; 7.7682x over previous
import jax
import jax.numpy as jnp
from jax.experimental import pallas as pl
from jax.experimental.pallas import tpu as pltpu

_MARGIN = 0.2
_EPS = 1e-6

_BM = 256
_BN = 512


def _body(a_ref, bt_ref, tcol_ref, trow_ref, num_ref, cnt_ref,
          btbf_ref, cp_ref, cn_ref, nacc_ref, cacc_ref):
    i = pl.program_id(0)
    nsteps = pl.num_programs(0)
    d_dim = a_ref.shape[1]
    b_dim = bt_ref.shape[1]

    @pl.when(i == 0)
    def _init():
        bt = bt_ref[...]
        btbf_ref[...] = bt.astype(jnp.bfloat16)
        colterm = jnp.sum(bt * (bt - (2.0 * _EPS)), axis=0, keepdims=True)
        colterm = colterm + (d_dim * _EPS * _EPS)
        posm = tcol_ref[...] == 1
        cp_ref[...] = jnp.where(posm, colterm, -jnp.inf).astype(jnp.bfloat16)
        cn_ref[...] = jnp.where(posm, jnp.inf, colterm).astype(jnp.bfloat16)
        nacc_ref[...] = jnp.zeros_like(nacc_ref)
        cacc_ref[...] = jnp.zeros_like(cacc_ref)

    a = a_ref[...]
    am2 = (a * -2.0).astype(jnp.bfloat16)
    rowterm = jnp.sum(a * (a + (2.0 * _EPS)), axis=1, keepdims=True)

    ninf = jnp.asarray(-jnp.inf, jnp.bfloat16)
    acc_p = jnp.full((_BM, 128), ninf, jnp.bfloat16)
    acc_n = jnp.full((_BM, 128), -ninf, jnp.bfloat16)
    for c in range(b_dim // _BN):
        btc = btbf_ref[:, c * _BN:(c + 1) * _BN]
        t2 = jax.lax.dot_general(am2, btc, (((1,), (0,)), ((), ())),
                                 preferred_element_type=jnp.float32
                                 ).astype(jnp.bfloat16)
        cp = cp_ref[0:1, c * _BN:(c + 1) * _BN]
        cn = cn_ref[0:1, c * _BN:(c + 1) * _BN]
        tp = t2 + cp
        tn = t2 + cn
        for s in range(_BN // 128):
            acc_p = jnp.maximum(acc_p, tp[:, s * 128:(s + 1) * 128])
            acc_n = jnp.minimum(acc_n, tn[:, s * 128:(s + 1) * 128])

    msp = jnp.max(acc_p, axis=1, keepdims=True).astype(jnp.float32) + rowterm
    msn = jnp.min(acc_n, axis=1, keepdims=True).astype(jnp.float32) + rowterm
    dp = jnp.sqrt(jnp.maximum(msp, 0.0))
    dn = jnp.sqrt(jnp.maximum(msn, 0.0))
    w = (trow_ref[...] == 1).astype(jnp.float32)
    nacc_ref[...] += jnp.maximum(dp - dn + _MARGIN, 0.0) * w
    cacc_ref[...] += w

    @pl.when(i == nsteps - 1)
    def _fin():
        num_ref[...] = jnp.sum(nacc_ref[...], keepdims=True)
        cnt_ref[...] = jnp.sum(cacc_ref[...], keepdims=True)


def kernel(emb1, emb2, target):
    b_dim, d_dim = emb1.shape
    nb = b_dim // _BM
    tgt = target.astype(jnp.int32)
    bt = emb2.T
    tcol = tgt.reshape(1, b_dim)
    trow = tgt.reshape(b_dim, 1)

    num, cnt = pl.pallas_call(
        _body,
        grid=(nb,),
        in_specs=[
            pl.BlockSpec((_BM, d_dim), lambda i: (i, 0)),
            pl.BlockSpec((d_dim, b_dim), lambda i: (0, 0)),
            pl.BlockSpec((1, b_dim), lambda i: (0, 0)),
            pl.BlockSpec((_BM, 1), lambda i: (i, 0)),
        ],
        out_specs=[
            pl.BlockSpec((1, 1), lambda i: (0, 0)),
            pl.BlockSpec((1, 1), lambda i: (0, 0)),
        ],
        out_shape=[
            jax.ShapeDtypeStruct((1, 1), jnp.float32),
            jax.ShapeDtypeStruct((1, 1), jnp.float32),
        ],
        scratch_shapes=[
            pltpu.VMEM((d_dim, b_dim), jnp.bfloat16),
            pltpu.VMEM((1, b_dim), jnp.bfloat16),
            pltpu.VMEM((1, b_dim), jnp.bfloat16),
            pltpu.VMEM((_BM, 1), jnp.float32),
            pltpu.VMEM((_BM, 1), jnp.float32),
        ],
        compiler_params=pltpu.CompilerParams(
            dimension_semantics=("arbitrary",),
            vmem_limit_bytes=48 * 1024 * 1024,
        ),
    )(emb1, bt, tcol, trow)

    return num[0, 0] / cnt[0, 0]


# speedup vs baseline: 8.3224x; 1.0713x over previous
import jax
import jax.numpy as jnp
from jax.experimental import pallas as pl
from jax.experimental.pallas import tpu as pltpu

_MARGIN = 0.2
_EPS = 1e-6

_BM = 512
_BN = 512


def _body(a_ref, bt_ref, tcol_ref, trow_ref, num_ref, cnt_ref,
          btbf_ref, cp_ref, cn_ref, nacc_ref, cacc_ref):
    i = pl.program_id(0)
    nsteps = pl.num_programs(0)
    d_dim = a_ref.shape[1]
    b_dim = bt_ref.shape[1]

    @pl.when(i == 0)
    def _init():
        bt = bt_ref[...]
        btbf_ref[...] = bt.astype(jnp.bfloat16)
        colterm = jnp.sum(bt * (bt - (2.0 * _EPS)), axis=0, keepdims=True)
        colterm = colterm + (d_dim * _EPS * _EPS)
        posm = tcol_ref[...] == 1
        cp_ref[...] = jnp.where(posm, colterm, -jnp.inf).astype(jnp.bfloat16)
        cn_ref[...] = jnp.where(posm, jnp.inf, colterm).astype(jnp.bfloat16)
        nacc_ref[...] = jnp.zeros_like(nacc_ref)
        cacc_ref[...] = jnp.zeros_like(cacc_ref)

    a = a_ref[...]
    am2 = (a * -2.0).astype(jnp.bfloat16)
    rowterm = jnp.sum(a * (a + (2.0 * _EPS)), axis=1, keepdims=True)

    ninf = jnp.asarray(-jnp.inf, jnp.bfloat16)
    acc_p = jnp.full((_BM, 128), ninf, jnp.bfloat16)
    acc_n = jnp.full((_BM, 128), -ninf, jnp.bfloat16)
    for c in range(b_dim // _BN):
        btc = btbf_ref[:, c * _BN:(c + 1) * _BN]
        t2 = jax.lax.dot_general(am2, btc, (((1,), (0,)), ((), ())),
                                 preferred_element_type=jnp.float32
                                 ).astype(jnp.bfloat16)
        cp = cp_ref[0:1, c * _BN:(c + 1) * _BN]
        cn = cn_ref[0:1, c * _BN:(c + 1) * _BN]
        tp = t2 + cp
        tn = t2 + cn
        for s in range(_BN // 128):
            acc_p = jnp.maximum(acc_p, tp[:, s * 128:(s + 1) * 128])
            acc_n = jnp.minimum(acc_n, tn[:, s * 128:(s + 1) * 128])

    msp = jnp.max(acc_p, axis=1, keepdims=True).astype(jnp.float32) + rowterm
    msn = jnp.min(acc_n, axis=1, keepdims=True).astype(jnp.float32) + rowterm
    dp = jnp.sqrt(jnp.maximum(msp, 0.0))
    dn = jnp.sqrt(jnp.maximum(msn, 0.0))
    w = (trow_ref[...] == 1).astype(jnp.float32)
    nacc_ref[...] += jnp.maximum(dp - dn + _MARGIN, 0.0) * w
    cacc_ref[...] += w

    @pl.when(i == nsteps - 1)
    def _fin():
        num_ref[...] = jnp.sum(nacc_ref[...], keepdims=True)
        cnt_ref[...] = jnp.sum(cacc_ref[...], keepdims=True)


def kernel(emb1, emb2, target):
    b_dim, d_dim = emb1.shape
    nb = b_dim // _BM
    tgt = target.astype(jnp.int32)
    bt = emb2.T
    tcol = tgt.reshape(1, b_dim)
    trow = tgt.reshape(b_dim, 1)

    num, cnt = pl.pallas_call(
        _body,
        grid=(nb,),
        in_specs=[
            pl.BlockSpec((_BM, d_dim), lambda i: (i, 0)),
            pl.BlockSpec((d_dim, b_dim), lambda i: (0, 0)),
            pl.BlockSpec((1, b_dim), lambda i: (0, 0)),
            pl.BlockSpec((_BM, 1), lambda i: (i, 0)),
        ],
        out_specs=[
            pl.BlockSpec((1, 1), lambda i: (0, 0)),
            pl.BlockSpec((1, 1), lambda i: (0, 0)),
        ],
        out_shape=[
            jax.ShapeDtypeStruct((1, 1), jnp.float32),
            jax.ShapeDtypeStruct((1, 1), jnp.float32),
        ],
        scratch_shapes=[
            pltpu.VMEM((d_dim, b_dim), jnp.bfloat16),
            pltpu.VMEM((1, b_dim), jnp.bfloat16),
            pltpu.VMEM((1, b_dim), jnp.bfloat16),
            pltpu.VMEM((_BM, 1), jnp.float32),
            pltpu.VMEM((_BM, 1), jnp.float32),
        ],
        compiler_params=pltpu.CompilerParams(
            dimension_semantics=("arbitrary",),
            vmem_limit_bytes=48 * 1024 * 1024,
        ),
    )(emb1, bt, tcol, trow)

    return num[0, 0] / cnt[0, 0]


# speedup vs baseline: 8.6887x; 1.0440x over previous
import jax
import jax.numpy as jnp
from jax.experimental import pallas as pl
from jax.experimental.pallas import tpu as pltpu

_MARGIN = 0.2
_EPS = 1e-6

_BM = 512
_BN = 512


def _body(a_ref, bt_ref, tcol_ref, trow_ref, num_ref, cnt_ref,
          btbf_ref, cp_ref, cn_ref, nacc_ref, cacc_ref):
    i = pl.program_id(0)
    nsteps = pl.num_programs(0)
    d_dim = a_ref.shape[1]
    b_dim = bt_ref.shape[1]

    @pl.when(i == 0)
    def _init():
        bt = bt_ref[...]
        btbf_ref[...] = bt.astype(jnp.bfloat16)
        colterm = jnp.sum(bt * (bt - (2.0 * _EPS)), axis=0, keepdims=True)
        colterm = colterm + (d_dim * _EPS * _EPS)
        posm = tcol_ref[...] == 1
        cp_ref[...] = jnp.where(posm, colterm, -jnp.inf).astype(jnp.bfloat16)
        cn_ref[...] = jnp.where(posm, jnp.inf, colterm).astype(jnp.bfloat16)
        nacc_ref[...] = jnp.zeros_like(nacc_ref)
        cacc_ref[...] = jnp.zeros_like(cacc_ref)

    a = a_ref[...]
    am2 = (a * -2.0).astype(jnp.bfloat16)
    rowterm = jnp.sum(a * (a + (2.0 * _EPS)), axis=1, keepdims=True)

    ninf = jnp.asarray(-jnp.inf, jnp.bfloat16)
    acc_p = jnp.full((_BM, 128), ninf, jnp.bfloat16)
    acc_n = jnp.full((_BM, 128), -ninf, jnp.bfloat16)
    for c in range(b_dim // _BN):
        btc = btbf_ref[:, c * _BN:(c + 1) * _BN]
        t2 = jax.lax.dot_general(am2, btc, (((1,), (0,)), ((), ())),
                                 preferred_element_type=jnp.float32
                                 ).astype(jnp.bfloat16)
        cp = cp_ref[0:1, c * _BN:(c + 1) * _BN]
        cn = cn_ref[0:1, c * _BN:(c + 1) * _BN]
        tp = t2 + cp
        tn = t2 + cn
        for s in range(_BN // 128):
            acc_p = jnp.maximum(acc_p, tp[:, s * 128:(s + 1) * 128])
            acc_n = jnp.minimum(acc_n, tn[:, s * 128:(s + 1) * 128])

    msp = jnp.max(acc_p, axis=1, keepdims=True).astype(jnp.float32) + rowterm
    msn = jnp.min(acc_n, axis=1, keepdims=True).astype(jnp.float32) + rowterm
    dp = jnp.sqrt(jnp.maximum(msp, 0.0))
    dn = jnp.sqrt(jnp.maximum(msn, 0.0))
    w = (trow_ref[...] == 1).astype(jnp.float32)
    nacc_ref[...] += jnp.maximum(dp - dn + _MARGIN, 0.0) * w
    cacc_ref[...] += w

    @pl.when(i == nsteps - 1)
    def _fin():
        num_ref[...] = jnp.sum(nacc_ref[...], keepdims=True)
        cnt_ref[...] = jnp.sum(cacc_ref[...], keepdims=True)


def kernel(emb1, emb2, target):
    b_dim, d_dim = emb1.shape
    nb = b_dim // _BM
    tgt = target.astype(jnp.int32)
    bt = emb2.T
    tcol = tgt.reshape(1, b_dim)
    trow = tgt.reshape(b_dim, 1)

    num, cnt = pl.pallas_call(
        _body,
        grid=(nb,),
        in_specs=[
            pl.BlockSpec((_BM, d_dim), lambda i: (i, 0)),
            pl.BlockSpec((d_dim, b_dim), lambda i: (0, 0)),
            pl.BlockSpec((1, b_dim), lambda i: (0, 0)),
            pl.BlockSpec((_BM, 1), lambda i: (i, 0)),
        ],
        out_specs=[
            pl.BlockSpec((1, 1), lambda i: (0, 0)),
            pl.BlockSpec((1, 1), lambda i: (0, 0)),
        ],
        out_shape=[
            jax.ShapeDtypeStruct((1, 1), jnp.float32),
            jax.ShapeDtypeStruct((1, 1), jnp.float32),
        ],
        scratch_shapes=[
            pltpu.VMEM((d_dim, b_dim), jnp.bfloat16),
            pltpu.VMEM((1, b_dim), jnp.bfloat16),
            pltpu.VMEM((1, b_dim), jnp.bfloat16),
            pltpu.VMEM((_BM, 1), jnp.float32),
            pltpu.VMEM((_BM, 1), jnp.float32),
        ],
        compiler_params=pltpu.CompilerParams(
            dimension_semantics=("arbitrary",),
            allow_input_fusion=(False, True, False, False),
            vmem_limit_bytes=48 * 1024 * 1024,
        ),
    )(emb1, bt, tcol, trow)

    return num[0, 0] / cnt[0, 0]


# speedup vs baseline: 8.8539x; 1.0190x over previous
import jax
import jax.numpy as jnp
from jax.experimental import pallas as pl
from jax.experimental.pallas import tpu as pltpu

_MARGIN = 0.2
_EPS = 1e-6

_BM = 512
_BN = 512


def _body(a_ref, bt_ref, tcol_ref, trow_ref, num_ref, cnt_ref,
          btbf_ref, cp_ref, cn_ref, nacc_ref, cacc_ref):
    i = pl.program_id(0)
    nsteps = pl.num_programs(0)
    d_dim = a_ref.shape[1]
    b_dim = bt_ref.shape[1]

    @pl.when(i == 0)
    def _init():
        bt = bt_ref[...]
        btbf_ref[...] = bt.astype(jnp.bfloat16)
        colterm = jnp.sum(bt * (bt - (2.0 * _EPS)), axis=0, keepdims=True)
        colterm = colterm + (d_dim * _EPS * _EPS)
        posm = tcol_ref[...] == 1
        cp_ref[...] = jnp.where(posm, colterm, -jnp.inf).astype(jnp.bfloat16)
        cn_ref[...] = jnp.where(posm, jnp.inf, colterm).astype(jnp.bfloat16)
        nacc_ref[...] = jnp.zeros_like(nacc_ref)
        cacc_ref[...] = jnp.zeros_like(cacc_ref)

    a = a_ref[...]
    am2 = (a * -2.0).astype(jnp.bfloat16)
    rowterm = jnp.sum(a * (a + (2.0 * _EPS)), axis=1, keepdims=True)

    ninf = jnp.asarray(-jnp.inf, jnp.bfloat16)
    acc_p = jnp.full((_BM, 128), ninf, jnp.bfloat16)
    acc_n = jnp.full((_BM, 128), -ninf, jnp.bfloat16)
    for c in range(b_dim // _BN):
        btc = btbf_ref[:, c * _BN:(c + 1) * _BN]
        t2 = jax.lax.dot_general(am2, btc, (((1,), (0,)), ((), ())),
                                 preferred_element_type=jnp.float32
                                 ).astype(jnp.bfloat16)
        cp = cp_ref[0:1, c * _BN:(c + 1) * _BN]
        cn = cn_ref[0:1, c * _BN:(c + 1) * _BN]
        tp = t2 + cp
        tn = t2 + cn
        for s in range(_BN // 128):
            acc_p = jnp.maximum(acc_p, tp[:, s * 128:(s + 1) * 128])
            acc_n = jnp.minimum(acc_n, tn[:, s * 128:(s + 1) * 128])

    msp = jnp.max(acc_p, axis=1, keepdims=True).astype(jnp.float32) + rowterm
    msn = jnp.min(acc_n, axis=1, keepdims=True).astype(jnp.float32) + rowterm
    dp = jnp.sqrt(jnp.maximum(msp, 0.0))
    dn = jnp.sqrt(jnp.maximum(msn, 0.0))
    w = (trow_ref[...] == 1).astype(jnp.float32)
    nacc_ref[...] += jnp.maximum(dp - dn + _MARGIN, 0.0) * w
    cacc_ref[...] += w

    @pl.when(i == nsteps - 1)
    def _fin():
        num_ref[...] = jnp.sum(nacc_ref[...], keepdims=True)
        cnt_ref[...] = jnp.sum(cacc_ref[...], keepdims=True)


def kernel(emb1, emb2, target):
    b_dim, d_dim = emb1.shape
    nb = b_dim // _BM
    tgt = target.astype(jnp.int32)
    bt = emb2.T
    tcol = tgt.reshape(1, b_dim)
    trow = tgt.reshape(b_dim, 1)

    num, cnt = pl.pallas_call(
        _body,
        grid=(nb,),
        in_specs=[
            pl.BlockSpec((_BM, d_dim), lambda i: (i, 0)),
            pl.BlockSpec((d_dim, b_dim), lambda i: (0, 0)),
            pl.BlockSpec((1, b_dim), lambda i: (0, 0)),
            pl.BlockSpec((_BM, 1), lambda i: (i, 0)),
        ],
        out_specs=[
            pl.BlockSpec((1, 1), lambda i: (0, 0)),
            pl.BlockSpec((1, 1), lambda i: (0, 0)),
        ],
        out_shape=[
            jax.ShapeDtypeStruct((1, 1), jnp.float32),
            jax.ShapeDtypeStruct((1, 1), jnp.float32),
        ],
        scratch_shapes=[
            pltpu.VMEM((d_dim, b_dim), jnp.bfloat16),
            pltpu.VMEM((1, b_dim), jnp.bfloat16),
            pltpu.VMEM((1, b_dim), jnp.bfloat16),
            pltpu.VMEM((_BM, 1), jnp.float32),
            pltpu.VMEM((_BM, 1), jnp.float32),
        ],
        compiler_params=pltpu.CompilerParams(
            dimension_semantics=("arbitrary",),
            allow_input_fusion=(False, True, True, True),
            vmem_limit_bytes=48 * 1024 * 1024,
        ),
    )(emb1, bt, tcol, trow)

    return num[0, 0] / cnt[0, 0]


# speedup vs baseline: 9.3963x; 1.0613x over previous
import jax
import jax.numpy as jnp
from jax.experimental import pallas as pl
from jax.experimental.pallas import tpu as pltpu

_MARGIN = 0.2
_EPS = 1e-6

_BM = 1024
_BN = 512


def _body(a_ref, bt_ref, tcol_ref, trow_ref, num_ref, cnt_ref,
          btbf_ref, cp_ref, cn_ref, nacc_ref, cacc_ref):
    i = pl.program_id(0)
    nsteps = pl.num_programs(0)
    d_dim = a_ref.shape[1]
    b_dim = bt_ref.shape[1]

    @pl.when(i == 0)
    def _init():
        bt = bt_ref[...]
        btbf_ref[...] = bt.astype(jnp.bfloat16)
        colterm = jnp.sum(bt * (bt - (2.0 * _EPS)), axis=0, keepdims=True)
        colterm = colterm + (d_dim * _EPS * _EPS)
        posm = tcol_ref[...] == 1
        cp_ref[...] = jnp.where(posm, colterm, -jnp.inf).astype(jnp.bfloat16)
        cn_ref[...] = jnp.where(posm, jnp.inf, colterm).astype(jnp.bfloat16)
        nacc_ref[...] = jnp.zeros_like(nacc_ref)
        cacc_ref[...] = jnp.zeros_like(cacc_ref)

    a = a_ref[...]
    am2 = (a * -2.0).astype(jnp.bfloat16)
    rowterm = jnp.sum(a * (a + (2.0 * _EPS)), axis=1, keepdims=True)

    ninf = jnp.asarray(-jnp.inf, jnp.bfloat16)
    acc_p = jnp.full((_BM, 128), ninf, jnp.bfloat16)
    acc_n = jnp.full((_BM, 128), -ninf, jnp.bfloat16)
    for c in range(b_dim // _BN):
        btc = btbf_ref[:, c * _BN:(c + 1) * _BN]
        t2 = jax.lax.dot_general(am2, btc, (((1,), (0,)), ((), ())),
                                 preferred_element_type=jnp.float32
                                 ).astype(jnp.bfloat16)
        cp = cp_ref[0:1, c * _BN:(c + 1) * _BN]
        cn = cn_ref[0:1, c * _BN:(c + 1) * _BN]
        tp = t2 + cp
        tn = t2 + cn
        for s in range(_BN // 128):
            acc_p = jnp.maximum(acc_p, tp[:, s * 128:(s + 1) * 128])
            acc_n = jnp.minimum(acc_n, tn[:, s * 128:(s + 1) * 128])

    msp = jnp.max(acc_p, axis=1, keepdims=True).astype(jnp.float32) + rowterm
    msn = jnp.min(acc_n, axis=1, keepdims=True).astype(jnp.float32) + rowterm
    dp = jnp.sqrt(jnp.maximum(msp, 0.0))
    dn = jnp.sqrt(jnp.maximum(msn, 0.0))
    w = (trow_ref[...] == 1).astype(jnp.float32)
    nacc_ref[...] += jnp.maximum(dp - dn + _MARGIN, 0.0) * w
    cacc_ref[...] += w

    @pl.when(i == nsteps - 1)
    def _fin():
        num_ref[...] = jnp.sum(nacc_ref[...], keepdims=True)
        cnt_ref[...] = jnp.sum(cacc_ref[...], keepdims=True)


def kernel(emb1, emb2, target):
    b_dim, d_dim = emb1.shape
    nb = b_dim // _BM
    tgt = target.astype(jnp.int32)
    bt = emb2.T
    tcol = tgt.reshape(1, b_dim)
    trow = tgt.reshape(b_dim, 1)

    num, cnt = pl.pallas_call(
        _body,
        grid=(nb,),
        in_specs=[
            pl.BlockSpec((_BM, d_dim), lambda i: (i, 0)),
            pl.BlockSpec((d_dim, b_dim), lambda i: (0, 0)),
            pl.BlockSpec((1, b_dim), lambda i: (0, 0)),
            pl.BlockSpec((_BM, 1), lambda i: (i, 0)),
        ],
        out_specs=[
            pl.BlockSpec((1, 1), lambda i: (0, 0)),
            pl.BlockSpec((1, 1), lambda i: (0, 0)),
        ],
        out_shape=[
            jax.ShapeDtypeStruct((1, 1), jnp.float32),
            jax.ShapeDtypeStruct((1, 1), jnp.float32),
        ],
        scratch_shapes=[
            pltpu.VMEM((d_dim, b_dim), jnp.bfloat16),
            pltpu.VMEM((1, b_dim), jnp.bfloat16),
            pltpu.VMEM((1, b_dim), jnp.bfloat16),
            pltpu.VMEM((_BM, 1), jnp.float32),
            pltpu.VMEM((_BM, 1), jnp.float32),
        ],
        compiler_params=pltpu.CompilerParams(
            dimension_semantics=("arbitrary",),
            allow_input_fusion=(False, True, True, True),
            vmem_limit_bytes=48 * 1024 * 1024,
        ),
    )(emb1, bt, tcol, trow)

    return num[0, 0] / cnt[0, 0]
